# baseline (speedup 1.0000x reference)
import functools

import jax
import jax.numpy as jnp
from jax import lax
from jax.experimental import pallas as pl
from jax.experimental.pallas import tpu as pltpu
from jax.experimental.pallas import tpu_sc as plsc

_N = 32768
_KNN = 20
_NG = 16
_QB = 1024
_CB = 1024
_NQB = _N // _QB
_NCB = _N // _CB
_BW = 128
_E = _N * _KNN
_GCH = 128
_BN = 512


def _knn_body(clo_ref, chi_ref, xq_ref, bq_ref, xc_ref, bc_ref, out_ref,
              wd_ref):
    qi = pl.program_id(0)
    xq = xq_ref[...]
    bq = bq_ref[0]
    qidx = qi * _QB + lax.broadcasted_iota(jnp.int32, (_QB, 1), 0)
    lanes = lax.broadcasted_iota(jnp.int32, (_QB, _BW), 1)
    lanesw = lax.broadcasted_iota(jnp.int32, (_QB, _BW + _CB), 1)
    imax = jnp.int32(2**31 - 1)
    best_d0 = jnp.full((_QB, _BW), jnp.inf, jnp.float32)
    best_i0 = lax.broadcasted_iota(jnp.int32, (_QB, _BW), 1)

    def cblock(cb, carry):
        bd, bi = carry
        xc = xc_ref[cb]
        bc = bc_ref[cb]
        cidx = cb * _CB + lax.broadcasted_iota(jnp.int32, (1, _CB), 1)
        t0 = xq[:, 0:1] - xc[0:1, :]
        acc = t0 * t0
        for d in range(1, 4):
            t = xq[:, d:d + 1] - xc[d:d + 1, :]
            acc = acc + t * t
        dmat = jnp.where(bq != bc, jnp.inf, acc)
        wd_ref[:, :_BW] = bd
        wd_ref[:, _BW:] = jnp.where(qidx == cidx, jnp.inf, dmat)

        def rnd(r, c):
            nd, ni = c
            wd = wd_ref[...]
            minv = jnp.min(wd, axis=1, keepdims=True)
            pos = jnp.argmin(wd, axis=1).reshape(_QB, 1).astype(jnp.int32)
            bi_val = jnp.min(jnp.where(lanes == pos, bi, imax),
                             axis=1, keepdims=True)
            idx = jnp.where(pos < _BW, bi_val, pos - _BW + cb * _CB)
            hit = lanes == r
            nd = jnp.where(hit, minv, nd)
            ni = jnp.where(hit, idx, ni)
            wd_ref[...] = jnp.where(lanesw == pos, jnp.inf, wd)
            return nd, ni

        return lax.fori_loop(0, _KNN, rnd, (best_d0, best_i0))

    _, best_i = lax.fori_loop(clo_ref[qi], chi_ref[qi], cblock,
                              (best_d0, best_i0))
    out_ref[...] = best_i[:, :32]


def _knn(xpad, batch):
    seg = jnp.searchsorted(batch, jnp.arange(_NG + 1, dtype=jnp.int32),
                           side="left").astype(jnp.int32)
    b2 = batch.reshape(_NQB, _QB)
    clo = seg[b2[:, 0]] // _CB
    chi = (seg[b2[:, -1] + 1] + _CB - 1) // _CB
    xc3 = xpad.T.reshape(16, _NCB, _CB).transpose(1, 0, 2)
    bq3 = batch.reshape(_NQB, _QB, 1)
    bc3 = batch.reshape(_NCB, 1, _CB)
    grid_spec = pltpu.PrefetchScalarGridSpec(
        num_scalar_prefetch=2,
        grid=(_NQB,),
        in_specs=[
            pl.BlockSpec((_QB, 16), lambda i, *_: (i, 0)),
            pl.BlockSpec((1, _QB, 1), lambda i, *_: (i, 0, 0)),
            pl.BlockSpec((_NCB, 16, _CB), lambda i, *_: (0, 0, 0)),
            pl.BlockSpec((_NCB, 1, _CB), lambda i, *_: (0, 0, 0)),
        ],
        out_specs=pl.BlockSpec((_QB, 32), lambda i, *_: (i, 0)),
        scratch_shapes=[
            pltpu.VMEM((_QB, _BW + _CB), jnp.float32),
        ],
    )
    nbr = pl.pallas_call(
        _knn_body,
        grid_spec=grid_spec,
        out_shape=jax.ShapeDtypeStruct((_N, 32), jnp.int32),
    )(clo, chi, xpad, bq3, xc3, bc3)
    return nbr[:, :_KNN]


def _gather_rows(table, idx):
    n_rows, dim = table.shape
    n_idx = idx.shape[0]
    grp = {16: 8, 64: 4, 128: 2}[dim]
    gr = grp * _GCH
    info = plsc.get_sparse_core_info()
    nw = info.num_cores * info.num_subcores
    per_w = n_idx // nw
    ng = per_w // gr
    mesh = plsc.VectorSubcoreMesh(core_axis_name="c", subcore_axis_name="s")

    @functools.partial(
        pl.kernel,
        out_type=jax.ShapeDtypeStruct((n_idx, dim), jnp.float32),
        mesh=mesh,
        scratch_types=[
            pltpu.VMEM((gr,), jnp.int32),
            pltpu.VMEM((gr,), jnp.int32),
            pltpu.VMEM((gr, dim), jnp.float32),
            pltpu.VMEM((gr, dim), jnp.float32),
            pltpu.SemaphoreType.DMA,
            pltpu.SemaphoreType.DMA,
        ],
        compiler_params=pltpu.CompilerParams(use_tc_tiling_on_sc=False),
    )
    def k(tab_hbm, idx_hbm, out_hbm, idx_a, idx_b, rows_a, rows_b,
          sem_a, sem_b):
        wid = lax.axis_index("s") * info.num_cores + lax.axis_index("c")
        base = wid * per_w

        def fire(g, idx_v, rows_v, sem):
            pltpu.sync_copy(idx_hbm.at[pl.ds(base + g * gr, gr)], idx_v)
            for b in range(grp):
                pltpu.async_copy(
                    tab_hbm.at[idx_v.at[pl.ds(b * _GCH, _GCH)]],
                    rows_v.at[pl.ds(b * _GCH, _GCH)], sem)

        def drain_store(g, idx_v, rows_v, sem):
            for b in range(grp):
                pltpu.make_async_copy(
                    tab_hbm.at[idx_v.at[pl.ds(b * _GCH, _GCH)]],
                    rows_v.at[pl.ds(b * _GCH, _GCH)], sem).wait()
            pltpu.sync_copy(rows_v, out_hbm.at[pl.ds(base + g * gr, gr)])

        fire(0, idx_a, rows_a, sem_a)

        def pair(p, _):
            g0 = 2 * p
            fire(g0 + 1, idx_b, rows_b, sem_b)
            drain_store(g0, idx_a, rows_a, sem_a)
            fire(g0 + 2, idx_a, rows_a, sem_a)
            drain_store(g0 + 1, idx_b, rows_b, sem_b)
            return 0

        lax.fori_loop(0, ng // 2 - 1, pair, 0)
        fire(ng - 1, idx_b, rows_b, sem_b)
        drain_store(ng - 2, idx_a, rows_a, sem_a)
        drain_store(ng - 1, idx_b, rows_b, sem_b)

    return k(table, idx)


def _edge_body(xi_ref, xg_ref, w1d_ref, w1b_ref, b1_ref, w2_ref, b2_ref,
               out_ref):
    ho = w2_ref.shape[1]
    ai = jnp.dot(xi_ref[...], w1d_ref[...],
                 preferred_element_type=jnp.float32) + b1_ref[...]
    xg = xg_ref[...].reshape(_KNN * _BN, xg_ref.shape[2])
    pj = jnp.dot(xg, w1b_ref[...], preferred_element_type=jnp.float32)
    pre = pj.reshape(_KNN, _BN, ho) + ai[None, :, :]
    h = jax.nn.relu(pre).reshape(_KNN * _BN, ho)
    h = jnp.dot(h, w2_ref[...], preferred_element_type=jnp.float32) \
        + b2_ref[...]
    out_ref[...] = jnp.max(h.reshape(_KNN, _BN, ho), axis=0)


def _edge_conv(x_nodes, xg3, w1, b1, w2, b2, hin_pad):
    hin = w1.shape[0] // 2
    ho = w1.shape[1]
    w1a, w1b = w1[:hin], w1[hin:]
    pad = ((0, hin_pad - hin), (0, 0))
    w1d = jnp.pad(w1a - w1b, pad)
    w1bp = jnp.pad(w1b, pad)
    grid = (_N // _BN,)
    out = pl.pallas_call(
        _edge_body,
        grid=grid,
        in_specs=[
            pl.BlockSpec((_BN, hin_pad), lambda i: (i, 0)),
            pl.BlockSpec((_KNN, _BN, hin_pad), lambda i: (0, i, 0)),
            pl.BlockSpec((hin_pad, ho), lambda i: (0, 0)),
            pl.BlockSpec((hin_pad, ho), lambda i: (0, 0)),
            pl.BlockSpec((1, ho), lambda i: (0, 0)),
            pl.BlockSpec((ho, ho), lambda i: (0, 0)),
            pl.BlockSpec((1, ho), lambda i: (0, 0)),
        ],
        out_specs=pl.BlockSpec((_BN, ho), lambda i: (i, 0)),
        out_shape=jax.ShapeDtypeStruct((_N, ho), jnp.float32),
    )(x_nodes, xg3, w1d, w1bp, b1.reshape(1, ho), w2, b2.reshape(1, ho))
    return out


def _final_body(x3_ref, bq_ref, w1_ref, b1_ref, w2_ref, b2_ref, w3_ref,
                b3_ref, out_ref, acc_ref):
    i = pl.program_id(0)

    @pl.when(i == 0)
    def _init():
        acc_ref[...] = jnp.full(acc_ref.shape, -jnp.inf, jnp.float32)

    xb = x3_ref[...]
    bq = bq_ref[0]
    parts = []
    for g in range(_NG):
        parts.append(jnp.max(jnp.where(bq == g, xb, -jnp.inf),
                             axis=0, keepdims=True))
    acc_ref[...] = jnp.maximum(acc_ref[...], jnp.concatenate(parts, axis=0))

    @pl.when(i == _NQB - 1)
    def _head():
        h = jax.nn.relu(jnp.dot(acc_ref[...], w1_ref[...],
                                preferred_element_type=jnp.float32)
                        + b1_ref[...])
        h = jax.nn.relu(jnp.dot(h, w2_ref[...],
                                preferred_element_type=jnp.float32)
                        + b2_ref[...])
        out_ref[...] = jnp.dot(h, w3_ref[...],
                               preferred_element_type=jnp.float32) \
            + b3_ref[...]


def _final(x3, batch, fc1w, fc1b, fc2w, fc2b, fc3w, fc3b):
    bq3 = batch.reshape(_NQB, _QB, 1)
    fc3wp = jnp.pad(fc3w, ((0, 0), (0, 128 - fc3w.shape[1])))
    fc3bp = jnp.pad(fc3b.reshape(1, -1), ((0, 0), (0, 128 - fc3b.shape[0])))
    out = pl.pallas_call(
        _final_body,
        grid=(_NQB,),
        in_specs=[
            pl.BlockSpec((_QB, 256), lambda i: (i, 0)),
            pl.BlockSpec((1, _QB, 1), lambda i: (i, 0, 0)),
            pl.BlockSpec((256, 512), lambda i: (0, 0)),
            pl.BlockSpec((1, 512), lambda i: (0, 0)),
            pl.BlockSpec((512, 256), lambda i: (0, 0)),
            pl.BlockSpec((1, 256), lambda i: (0, 0)),
            pl.BlockSpec((256, 128), lambda i: (0, 0)),
            pl.BlockSpec((1, 128), lambda i: (0, 0)),
        ],
        out_specs=pl.BlockSpec((_NG, 128), lambda i: (0, 0)),
        out_shape=jax.ShapeDtypeStruct((_NG, 128), jnp.float32),
        scratch_shapes=[pltpu.VMEM((_NG, 256), jnp.float32)],
    )(x3, bq3, fc1w, fc1b.reshape(1, -1), fc2w, fc2b.reshape(1, -1),
      fc3wp, fc3bp)
    return out[:, :1]


def kernel(x, batch, c1w1, c1b1, c1w2, c1b2, c2w1, c2b1, c2w2, c2b2,
           c3w1, c3b1, c3w2, c3b2, fc1w, fc1b, fc2w, fc2b, fc3w, fc3b):
    batch = batch.astype(jnp.int32)
    xpad = jnp.pad(x, ((0, 0), (0, 12)))

    nbr = _knn(xpad, batch)
    src_t = nbr.T.reshape(-1)

    xg1 = _gather_rows(xpad, src_t).reshape(_KNN, _N, 16)
    x1 = _edge_conv(xpad, xg1, c1w1, c1b1, c1w2, c1b2, 16)

    xg2 = _gather_rows(x1, src_t).reshape(_KNN, _N, 64)
    x2 = _edge_conv(x1, xg2, c2w1, c2b1, c2w2, c2b2, 64)

    xg3 = _gather_rows(x2, src_t).reshape(_KNN, _N, 128)
    x3 = _edge_conv(x2, xg3, c3w1, c3b1, c3w2, c3b2, 128)

    return _final(x3, batch, fc1w, fc1b, fc2w, fc2b, fc3w, fc3b)


# speedup vs baseline: 1.3489x; 1.3489x over previous
import functools

import jax
import jax.numpy as jnp
from jax import lax
from jax.experimental import pallas as pl
from jax.experimental.pallas import tpu as pltpu
from jax.experimental.pallas import tpu_sc as plsc

_N = 32768
_KNN = 20
_NG = 16
_QB = 1024
_CB = 1024
_NQB = _N // _QB
_NCB = _N // _CB
_BW = 128
_E = _N * _KNN
_GCH = 128
_BN = 512


def _knn_body(clo_ref, chi_ref, xq_ref, bq_ref, xc_ref, bc_ref, out_ref,
              wd_ref, wi_ref):
    qi = pl.program_id(0)
    xq = xq_ref[...]
    bq = bq_ref[0]
    qidx = qi * _QB + lax.broadcasted_iota(jnp.int32, (_QB, 1), 0)
    lanes = lax.broadcasted_iota(jnp.int32, (_QB, _BW), 1)
    lanesw = lax.broadcasted_iota(jnp.int32, (_QB, _BW + _CB), 1)
    imax = jnp.int32(2**31 - 1)
    best_d0 = jnp.full((_QB, _BW), jnp.inf, jnp.float32)
    best_i0 = lax.broadcasted_iota(jnp.int32, (_QB, _BW), 1)

    def cblock(cb, carry):
        bd, bi = carry
        xc = xc_ref[cb]
        bc = bc_ref[cb]
        cidx = cb * _CB + lax.broadcasted_iota(jnp.int32, (1, _CB), 1)
        t0 = xq[:, 0:1] - xc[0:1, :]
        acc = t0 * t0
        for d in range(1, 4):
            t = xq[:, d:d + 1] - xc[d:d + 1, :]
            acc = acc + t * t
        dmat = jnp.where(bq != bc, jnp.inf, acc)
        wd_ref[:, :_BW] = bd
        wd_ref[:, _BW:] = jnp.where(qidx == cidx, jnp.inf, dmat)
        wi_ref[:, :_BW] = bi
        wi_ref[:, _BW:] = jnp.broadcast_to(cidx, (_QB, _CB))

        def rnd(r, c):
            nd, ni = c
            wd = wd_ref[...]
            wi = wi_ref[...]
            minv = jnp.min(wd, axis=1, keepdims=True)
            sel = jnp.min(jnp.where(wd == minv, wi, imax),
                          axis=1, keepdims=True)
            hit = lanes == r
            nd = jnp.where(hit, minv, nd)
            ni = jnp.where(hit, sel, ni)
            wd_ref[...] = jnp.where(wi == sel, jnp.inf, wd)
            return nd, ni

        return lax.fori_loop(0, _KNN, rnd, (best_d0, best_i0))

    _, best_i = lax.fori_loop(clo_ref[qi], chi_ref[qi], cblock,
                              (best_d0, best_i0))
    out_ref[...] = best_i[:, :32]


def _knn(xpad, batch):
    seg = jnp.searchsorted(batch, jnp.arange(_NG + 1, dtype=jnp.int32),
                           side="left").astype(jnp.int32)
    b2 = batch.reshape(_NQB, _QB)
    clo = seg[b2[:, 0]] // _CB
    chi = (seg[b2[:, -1] + 1] + _CB - 1) // _CB
    xc3 = xpad.T.reshape(16, _NCB, _CB).transpose(1, 0, 2)
    bq3 = batch.reshape(_NQB, _QB, 1)
    bc3 = batch.reshape(_NCB, 1, _CB)
    grid_spec = pltpu.PrefetchScalarGridSpec(
        num_scalar_prefetch=2,
        grid=(_NQB,),
        in_specs=[
            pl.BlockSpec((_QB, 16), lambda i, *_: (i, 0)),
            pl.BlockSpec((1, _QB, 1), lambda i, *_: (i, 0, 0)),
            pl.BlockSpec((_NCB, 16, _CB), lambda i, *_: (0, 0, 0)),
            pl.BlockSpec((_NCB, 1, _CB), lambda i, *_: (0, 0, 0)),
        ],
        out_specs=pl.BlockSpec((_QB, 32), lambda i, *_: (i, 0)),
        scratch_shapes=[
            pltpu.VMEM((_QB, _BW + _CB), jnp.float32),
            pltpu.VMEM((_QB, _BW + _CB), jnp.int32),
        ],
    )
    nbr = pl.pallas_call(
        _knn_body,
        grid_spec=grid_spec,
        out_shape=jax.ShapeDtypeStruct((_N, 32), jnp.int32),
    )(clo, chi, xpad, bq3, xc3, bc3)
    return nbr[:, :_KNN]


def _gather_rows(table, idx):
    n_rows, dim = table.shape
    n_idx = idx.shape[0]
    grp = (128 * 1024) // (_GCH * dim * table.dtype.itemsize)
    gr = grp * _GCH
    info = plsc.get_sparse_core_info()
    nw = info.num_cores * info.num_subcores
    per_w = n_idx // nw
    ng = per_w // gr
    mesh = plsc.VectorSubcoreMesh(core_axis_name="c", subcore_axis_name="s")

    @functools.partial(
        pl.kernel,
        out_type=jax.ShapeDtypeStruct((n_idx, dim), table.dtype),
        mesh=mesh,
        scratch_types=[
            pltpu.VMEM((gr,), jnp.int32),
            pltpu.VMEM((gr,), jnp.int32),
            pltpu.VMEM((gr, dim), table.dtype),
            pltpu.VMEM((gr, dim), table.dtype),
            pltpu.SemaphoreType.DMA,
            pltpu.SemaphoreType.DMA,
        ],
        compiler_params=pltpu.CompilerParams(use_tc_tiling_on_sc=False),
    )
    def k(tab_hbm, idx_hbm, out_hbm, idx_a, idx_b, rows_a, rows_b,
          sem_a, sem_b):
        wid = lax.axis_index("s") * info.num_cores + lax.axis_index("c")
        base = wid * per_w

        def fire(g, idx_v, rows_v, sem):
            pltpu.sync_copy(idx_hbm.at[pl.ds(base + g * gr, gr)], idx_v)
            for b in range(grp):
                pltpu.async_copy(
                    tab_hbm.at[idx_v.at[pl.ds(b * _GCH, _GCH)]],
                    rows_v.at[pl.ds(b * _GCH, _GCH)], sem)

        def drain_store(g, idx_v, rows_v, sem):
            for b in range(grp):
                pltpu.make_async_copy(
                    tab_hbm.at[idx_v.at[pl.ds(b * _GCH, _GCH)]],
                    rows_v.at[pl.ds(b * _GCH, _GCH)], sem).wait()
            pltpu.sync_copy(rows_v, out_hbm.at[pl.ds(base + g * gr, gr)])

        fire(0, idx_a, rows_a, sem_a)

        def pair(p, _):
            g0 = 2 * p
            fire(g0 + 1, idx_b, rows_b, sem_b)
            drain_store(g0, idx_a, rows_a, sem_a)
            fire(g0 + 2, idx_a, rows_a, sem_a)
            drain_store(g0 + 1, idx_b, rows_b, sem_b)
            return 0

        lax.fori_loop(0, ng // 2 - 1, pair, 0)
        fire(ng - 1, idx_b, rows_b, sem_b)
        drain_store(ng - 2, idx_a, rows_a, sem_a)
        drain_store(ng - 1, idx_b, rows_b, sem_b)

    return k(table, idx)


def _edge_body(xi_ref, xg_ref, w1d_ref, w1b_ref, b1_ref, w2_ref, b2_ref,
               out_ref):
    ho = w2_ref.shape[1]
    ai = jnp.dot(xi_ref[...], w1d_ref[...],
                 preferred_element_type=jnp.float32) + b1_ref[...]
    xg = xg_ref[...].reshape(_KNN * _BN, xg_ref.shape[2])
    pj = jnp.dot(xg, w1b_ref[...], preferred_element_type=jnp.float32)
    pre = pj.reshape(_KNN, _BN, ho) + ai[None, :, :]
    h = jax.nn.relu(pre).reshape(_KNN * _BN, ho)
    h = jnp.dot(h, w2_ref[...], preferred_element_type=jnp.float32) \
        + b2_ref[...]
    out_ref[...] = jnp.max(h.reshape(_KNN, _BN, ho), axis=0)


def _edge_conv(x_nodes, xg3, w1, b1, w2, b2, hin_pad):
    hin = w1.shape[0] // 2
    ho = w1.shape[1]
    w1a, w1b = w1[:hin], w1[hin:]
    pad = ((0, hin_pad - hin), (0, 0))
    w1d = jnp.pad(w1a - w1b, pad)
    w1bp = jnp.pad(w1b, pad)
    grid = (_N // _BN,)
    out = pl.pallas_call(
        _edge_body,
        grid=grid,
        in_specs=[
            pl.BlockSpec((_BN, hin_pad), lambda i: (i, 0)),
            pl.BlockSpec((_KNN, _BN, hin_pad), lambda i: (0, i, 0)),
            pl.BlockSpec((hin_pad, ho), lambda i: (0, 0)),
            pl.BlockSpec((hin_pad, ho), lambda i: (0, 0)),
            pl.BlockSpec((1, ho), lambda i: (0, 0)),
            pl.BlockSpec((ho, ho), lambda i: (0, 0)),
            pl.BlockSpec((1, ho), lambda i: (0, 0)),
        ],
        out_specs=pl.BlockSpec((_BN, ho), lambda i: (i, 0)),
        out_shape=jax.ShapeDtypeStruct((_N, ho), jnp.float32),
    )(x_nodes, xg3, w1d, w1bp, b1.reshape(1, ho), w2, b2.reshape(1, ho))
    return out


def _final_body(x3_ref, bq_ref, w1_ref, b1_ref, w2_ref, b2_ref, w3_ref,
                b3_ref, out_ref, acc_ref):
    i = pl.program_id(0)

    @pl.when(i == 0)
    def _init():
        acc_ref[...] = jnp.full(acc_ref.shape, -jnp.inf, jnp.float32)

    xb = x3_ref[...]
    bq = bq_ref[0]
    parts = []
    for g in range(_NG):
        parts.append(jnp.max(jnp.where(bq == g, xb, -jnp.inf),
                             axis=0, keepdims=True))
    acc_ref[...] = jnp.maximum(acc_ref[...], jnp.concatenate(parts, axis=0))

    @pl.when(i == _NQB - 1)
    def _head():
        h = jax.nn.relu(jnp.dot(acc_ref[...], w1_ref[...],
                                preferred_element_type=jnp.float32)
                        + b1_ref[...])
        h = jax.nn.relu(jnp.dot(h, w2_ref[...],
                                preferred_element_type=jnp.float32)
                        + b2_ref[...])
        out_ref[...] = jnp.dot(h, w3_ref[...],
                               preferred_element_type=jnp.float32) \
            + b3_ref[...]


def _final(x3, batch, fc1w, fc1b, fc2w, fc2b, fc3w, fc3b):
    bq3 = batch.reshape(_NQB, _QB, 1)
    fc3wp = jnp.pad(fc3w, ((0, 0), (0, 128 - fc3w.shape[1])))
    fc3bp = jnp.pad(fc3b.reshape(1, -1), ((0, 0), (0, 128 - fc3b.shape[0])))
    out = pl.pallas_call(
        _final_body,
        grid=(_NQB,),
        in_specs=[
            pl.BlockSpec((_QB, 256), lambda i: (i, 0)),
            pl.BlockSpec((1, _QB, 1), lambda i: (i, 0, 0)),
            pl.BlockSpec((256, 512), lambda i: (0, 0)),
            pl.BlockSpec((1, 512), lambda i: (0, 0)),
            pl.BlockSpec((512, 256), lambda i: (0, 0)),
            pl.BlockSpec((1, 256), lambda i: (0, 0)),
            pl.BlockSpec((256, 128), lambda i: (0, 0)),
            pl.BlockSpec((1, 128), lambda i: (0, 0)),
        ],
        out_specs=pl.BlockSpec((_NG, 128), lambda i: (0, 0)),
        out_shape=jax.ShapeDtypeStruct((_NG, 128), jnp.float32),
        scratch_shapes=[pltpu.VMEM((_NG, 256), jnp.float32)],
    )(x3, bq3, fc1w, fc1b.reshape(1, -1), fc2w, fc2b.reshape(1, -1),
      fc3wp, fc3bp)
    return out[:, :1]


def kernel(x, batch, c1w1, c1b1, c1w2, c1b2, c2w1, c2b1, c2w2, c2b2,
           c3w1, c3b1, c3w2, c3b2, fc1w, fc1b, fc2w, fc2b, fc3w, fc3b):
    batch = batch.astype(jnp.int32)
    xpad = jnp.pad(x, ((0, 0), (0, 12)))

    nbr = _knn(xpad, batch)
    src_t = nbr.T.reshape(-1)

    xg1 = _gather_rows(xpad, src_t).reshape(_KNN, _N, 16)
    x1 = _edge_conv(xpad, xg1, c1w1, c1b1, c1w2, c1b2, 16)

    xg2 = _gather_rows(x1, src_t).reshape(_KNN, _N, 64)
    x2 = _edge_conv(x1, xg2, c2w1, c2b1, c2w2, c2b2, 64)

    xg3 = _gather_rows(x2, src_t).reshape(_KNN, _N, 128)
    x3 = _edge_conv(x2, xg3, c3w1, c3b1, c3w2, c3b2, 128)

    return _final(x3, batch, fc1w, fc1b, fc2w, fc2b, fc3w, fc3b)


# speedup vs baseline: 1.3530x; 1.0031x over previous
import functools

import jax
import jax.numpy as jnp
from jax import lax
from jax.experimental import pallas as pl
from jax.experimental.pallas import tpu as pltpu
from jax.experimental.pallas import tpu_sc as plsc

_N = 32768
_KNN = 20
_NG = 16
_QB = 1024
_CB = 1024
_NQB = _N // _QB
_NCB = _N // _CB
_BW = 128
_E = _N * _KNN
_GCH = 128
_BN = 512


def _knn_body(clo_ref, chi_ref, xq_ref, bq_ref, xc_ref, bc_ref, out_ref,
              wd_ref, wi_ref):
    qi = pl.program_id(0)
    xq = xq_ref[...]
    bq = bq_ref[0]
    qidx = qi * _QB + lax.broadcasted_iota(jnp.int32, (_QB, 1), 0)
    lanes = lax.broadcasted_iota(jnp.int32, (_QB, _BW), 1)
    lanesw = lax.broadcasted_iota(jnp.int32, (_QB, _BW + _CB), 1)
    imax = jnp.int32(2**31 - 1)
    best_d0 = jnp.full((_QB, _BW), jnp.inf, jnp.float32)
    best_i0 = lax.broadcasted_iota(jnp.int32, (_QB, _BW), 1)

    def cblock(cb, carry):
        bd, bi = carry
        xc = xc_ref[cb]
        bc = bc_ref[cb]
        cidx = cb * _CB + lax.broadcasted_iota(jnp.int32, (1, _CB), 1)
        t0 = xq[:, 0:1] - xc[0:1, :]
        acc = t0 * t0
        for d in range(1, 4):
            t = xq[:, d:d + 1] - xc[d:d + 1, :]
            acc = acc + t * t
        dmat = jnp.where(bq != bc, jnp.inf, acc)
        wd_ref[:, :_BW] = bd
        wd_ref[:, _BW:] = jnp.where(qidx == cidx, jnp.inf, dmat)
        wi_ref[:, :_BW] = bi
        wi_ref[:, _BW:] = jnp.broadcast_to(cidx, (_QB, _CB))

        def rnd(r, c):
            nd, ni = c
            wd = wd_ref[...]
            wi = wi_ref[...]
            minv = jnp.min(wd, axis=1, keepdims=True)
            sel = jnp.min(jnp.where(wd == minv, wi, imax),
                          axis=1, keepdims=True)
            hit = lanes == r
            nd = jnp.where(hit, minv, nd)
            ni = jnp.where(hit, sel, ni)
            wd_ref[...] = jnp.where(wi == sel, jnp.inf, wd)
            return nd, ni

        return lax.fori_loop(0, _KNN, rnd, (best_d0, best_i0))

    _, best_i = lax.fori_loop(clo_ref[qi], chi_ref[qi], cblock,
                              (best_d0, best_i0))
    out_ref[...] = best_i[:, :32]


def _knn(xpad, batch):
    seg = jnp.searchsorted(batch, jnp.arange(_NG + 1, dtype=jnp.int32),
                           side="left").astype(jnp.int32)
    b2 = batch.reshape(_NQB, _QB)
    clo = seg[b2[:, 0]] // _CB
    chi = (seg[b2[:, -1] + 1] + _CB - 1) // _CB
    xc3 = xpad.T.reshape(16, _NCB, _CB).transpose(1, 0, 2)
    bq3 = batch.reshape(_NQB, _QB, 1)
    bc3 = batch.reshape(_NCB, 1, _CB)
    grid_spec = pltpu.PrefetchScalarGridSpec(
        num_scalar_prefetch=2,
        grid=(_NQB,),
        in_specs=[
            pl.BlockSpec((_QB, 16), lambda i, *_: (i, 0)),
            pl.BlockSpec((1, _QB, 1), lambda i, *_: (i, 0, 0)),
            pl.BlockSpec((_NCB, 16, _CB), lambda i, *_: (0, 0, 0)),
            pl.BlockSpec((_NCB, 1, _CB), lambda i, *_: (0, 0, 0)),
        ],
        out_specs=pl.BlockSpec((_QB, 32), lambda i, *_: (i, 0)),
        scratch_shapes=[
            pltpu.VMEM((_QB, _BW + _CB), jnp.float32),
            pltpu.VMEM((_QB, _BW + _CB), jnp.int32),
        ],
    )
    nbr = pl.pallas_call(
        _knn_body,
        grid_spec=grid_spec,
        out_shape=jax.ShapeDtypeStruct((_N, 32), jnp.int32),
    )(clo, chi, xpad, bq3, xc3, bc3)
    return nbr[:, :_KNN]


def _gather_rows(table, idx):
    n_rows, dim = table.shape
    n_idx = idx.shape[0]
    grp = (128 * 1024) // (_GCH * dim * table.dtype.itemsize)
    gr = grp * _GCH
    info = plsc.get_sparse_core_info()
    nw = info.num_cores * info.num_subcores
    per_w = n_idx // nw
    ng = per_w // gr
    mesh = plsc.VectorSubcoreMesh(core_axis_name="c", subcore_axis_name="s")

    @functools.partial(
        pl.kernel,
        out_type=jax.ShapeDtypeStruct((n_idx, dim), table.dtype),
        mesh=mesh,
        scratch_types=[
            pltpu.VMEM((gr,), jnp.int32),
            pltpu.VMEM((gr,), jnp.int32),
            pltpu.VMEM((gr, dim), table.dtype),
            pltpu.VMEM((gr, dim), table.dtype),
            pltpu.SemaphoreType.DMA,
            pltpu.SemaphoreType.DMA,
        ],
        compiler_params=pltpu.CompilerParams(use_tc_tiling_on_sc=False),
    )
    def k(tab_hbm, idx_hbm, out_hbm, idx_a, idx_b, rows_a, rows_b,
          sem_a, sem_b):
        wid = lax.axis_index("s") * info.num_cores + lax.axis_index("c")
        base = wid * per_w

        def fire(g, idx_v, rows_v, sem):
            pltpu.sync_copy(idx_hbm.at[pl.ds(base + g * gr, gr)], idx_v)
            for b in range(grp):
                pltpu.async_copy(
                    tab_hbm.at[idx_v.at[pl.ds(b * _GCH, _GCH)]],
                    rows_v.at[pl.ds(b * _GCH, _GCH)], sem)

        def drain_store(g, idx_v, rows_v, sem):
            for b in range(grp):
                pltpu.make_async_copy(
                    tab_hbm.at[idx_v.at[pl.ds(b * _GCH, _GCH)]],
                    rows_v.at[pl.ds(b * _GCH, _GCH)], sem).wait()
            pltpu.sync_copy(rows_v, out_hbm.at[pl.ds(base + g * gr, gr)])

        fire(0, idx_a, rows_a, sem_a)

        def pair(p, _):
            g0 = 2 * p
            fire(g0 + 1, idx_b, rows_b, sem_b)
            drain_store(g0, idx_a, rows_a, sem_a)
            fire(g0 + 2, idx_a, rows_a, sem_a)
            drain_store(g0 + 1, idx_b, rows_b, sem_b)
            return 0

        if ng % 2 == 0:
            lax.fori_loop(0, ng // 2 - 1, pair, 0)
            fire(ng - 1, idx_b, rows_b, sem_b)
            drain_store(ng - 2, idx_a, rows_a, sem_a)
            drain_store(ng - 1, idx_b, rows_b, sem_b)
        else:
            lax.fori_loop(0, ng // 2, pair, 0)
            drain_store(ng - 1, idx_a, rows_a, sem_a)

    return k(table, idx)


def _edge_body(xi_ref, xg_ref, w1d_ref, w1b_ref, b1_ref, w2_ref, b2_ref,
               out_ref):
    ho = w2_ref.shape[1]
    ai = jnp.dot(xi_ref[...], w1d_ref[...],
                 preferred_element_type=jnp.float32) + b1_ref[...]
    xg = xg_ref[...].reshape(_KNN * _BN, xg_ref.shape[2])
    pj = jnp.dot(xg, w1b_ref[...], preferred_element_type=jnp.float32)
    pre = pj.reshape(_KNN, _BN, ho) + ai[None, :, :]
    h = jax.nn.relu(pre).reshape(_KNN * _BN, ho)
    h = jnp.dot(h, w2_ref[...], preferred_element_type=jnp.float32) \
        + b2_ref[...]
    out_ref[...] = jnp.max(h.reshape(_KNN, _BN, ho), axis=0)


def _edge_conv(x_nodes, xg3, w1, b1, w2, b2, hin_pad, blk_off, n_half):
    hin = w1.shape[0] // 2
    ho = w1.shape[1]
    w1a, w1b = w1[:hin], w1[hin:]
    pad = ((0, hin_pad - hin), (0, 0))
    w1d = jnp.pad(w1a - w1b, pad)
    w1bp = jnp.pad(w1b, pad)
    grid = (n_half // _BN,)
    out = pl.pallas_call(
        _edge_body,
        grid=grid,
        in_specs=[
            pl.BlockSpec((_BN, hin_pad), lambda i: (i + blk_off, 0)),
            pl.BlockSpec((_KNN, _BN, hin_pad), lambda i: (0, i, 0)),
            pl.BlockSpec((hin_pad, ho), lambda i: (0, 0)),
            pl.BlockSpec((hin_pad, ho), lambda i: (0, 0)),
            pl.BlockSpec((1, ho), lambda i: (0, 0)),
            pl.BlockSpec((ho, ho), lambda i: (0, 0)),
            pl.BlockSpec((1, ho), lambda i: (0, 0)),
        ],
        out_specs=pl.BlockSpec((_BN, ho), lambda i: (i, 0)),
        out_shape=jax.ShapeDtypeStruct((n_half, ho), jnp.float32),
    )(x_nodes, xg3, w1d, w1bp, b1.reshape(1, ho), w2, b2.reshape(1, ho))
    return out


def _edge_layer(x_nodes, nbr, w1, b1, w2, b2, hin_pad):
    nh = _N // 2
    src_a = nbr[:nh].T.reshape(-1)
    src_b = nbr[nh:].T.reshape(-1)
    xg_a = _gather_rows(x_nodes, src_a).reshape(_KNN, nh, hin_pad)
    xg_b = _gather_rows(x_nodes, src_b).reshape(_KNN, nh, hin_pad)
    ya = _edge_conv(x_nodes, xg_a, w1, b1, w2, b2, hin_pad, 0, nh)
    yb = _edge_conv(x_nodes, xg_b, w1, b1, w2, b2, hin_pad,
                    nh // _BN, nh)
    return jnp.concatenate([ya, yb], axis=0)


def _final_body(x3_ref, bq_ref, w1_ref, b1_ref, w2_ref, b2_ref, w3_ref,
                b3_ref, out_ref, acc_ref):
    i = pl.program_id(0)

    @pl.when(i == 0)
    def _init():
        acc_ref[...] = jnp.full(acc_ref.shape, -jnp.inf, jnp.float32)

    xb = x3_ref[...]
    bq = bq_ref[0]
    parts = []
    for g in range(_NG):
        parts.append(jnp.max(jnp.where(bq == g, xb, -jnp.inf),
                             axis=0, keepdims=True))
    acc_ref[...] = jnp.maximum(acc_ref[...], jnp.concatenate(parts, axis=0))

    @pl.when(i == _NQB - 1)
    def _head():
        h = jax.nn.relu(jnp.dot(acc_ref[...], w1_ref[...],
                                preferred_element_type=jnp.float32)
                        + b1_ref[...])
        h = jax.nn.relu(jnp.dot(h, w2_ref[...],
                                preferred_element_type=jnp.float32)
                        + b2_ref[...])
        out_ref[...] = jnp.dot(h, w3_ref[...],
                               preferred_element_type=jnp.float32) \
            + b3_ref[...]


def _final(x3, batch, fc1w, fc1b, fc2w, fc2b, fc3w, fc3b):
    bq3 = batch.reshape(_NQB, _QB, 1)
    fc3wp = jnp.pad(fc3w, ((0, 0), (0, 128 - fc3w.shape[1])))
    fc3bp = jnp.pad(fc3b.reshape(1, -1), ((0, 0), (0, 128 - fc3b.shape[0])))
    out = pl.pallas_call(
        _final_body,
        grid=(_NQB,),
        in_specs=[
            pl.BlockSpec((_QB, 256), lambda i: (i, 0)),
            pl.BlockSpec((1, _QB, 1), lambda i: (i, 0, 0)),
            pl.BlockSpec((256, 512), lambda i: (0, 0)),
            pl.BlockSpec((1, 512), lambda i: (0, 0)),
            pl.BlockSpec((512, 256), lambda i: (0, 0)),
            pl.BlockSpec((1, 256), lambda i: (0, 0)),
            pl.BlockSpec((256, 128), lambda i: (0, 0)),
            pl.BlockSpec((1, 128), lambda i: (0, 0)),
        ],
        out_specs=pl.BlockSpec((_NG, 128), lambda i: (0, 0)),
        out_shape=jax.ShapeDtypeStruct((_NG, 128), jnp.float32),
        scratch_shapes=[pltpu.VMEM((_NG, 256), jnp.float32)],
    )(x3, bq3, fc1w, fc1b.reshape(1, -1), fc2w, fc2b.reshape(1, -1),
      fc3wp, fc3bp)
    return out[:, :1]


def kernel(x, batch, c1w1, c1b1, c1w2, c1b2, c2w1, c2b1, c2w2, c2b2,
           c3w1, c3b1, c3w2, c3b2, fc1w, fc1b, fc2w, fc2b, fc3w, fc3b):
    batch = batch.astype(jnp.int32)
    xpad = jnp.pad(x, ((0, 0), (0, 12)))

    nbr = _knn(xpad, batch)
    x1 = _edge_layer(xpad, nbr, c1w1, c1b1, c1w2, c1b2, 16)
    x2 = _edge_layer(x1, nbr, c2w1, c2b1, c2w2, c2b2, 64)
    x3 = _edge_layer(x2, nbr, c3w1, c3b1, c3w2, c3b2, 128)

    return _final(x3, batch, fc1w, fc1b, fc2w, fc2b, fc3w, fc3b)
